# baseline (speedup 1.0000x reference)
import functools

import jax
import jax.numpy as jnp
from jax import lax
from jax.experimental import pallas as pl
from jax.experimental.pallas import tpu as pltpu
from jax.experimental.pallas import tpu_sc as plsc

_VOCAB = 100000
_DIM = 64
_BATCH = 1024
_MAX_NORM = 1.0

_NUM_CORES = 2
_NUM_SUBCORES = 16
_NW = _NUM_CORES * _NUM_SUBCORES
_BPW = _BATCH // _NW

_V_BLK = 2048
_GRID = (_VOCAB + _V_BLK - 1) // _V_BLK
_TAIL = _VOCAB - (_GRID - 1) * _V_BLK
_NBUF = 4
_NCHUNK = 8
_ROWS = _BATCH // _NCHUNK
_NTHREAD = 2

_sc_gather_fn = None


def _get_sc_gather():
    global _sc_gather_fn
    if _sc_gather_fn is None:
        mesh = plsc.VectorSubcoreMesh(core_axis_name="c", subcore_axis_name="s")

        @functools.partial(
            pl.kernel,
            mesh=mesh,
            compiler_params=pltpu.CompilerParams(use_tc_tiling_on_sc=False),
            out_type=jax.ShapeDtypeStruct((_BATCH, _DIM), jnp.float32),
            scratch_types=[
                pltpu.VMEM((_BPW,), jnp.int32),
                pltpu.VMEM((_BPW, _DIM), jnp.float32),
                pltpu.SemaphoreType.DMA,
            ],
        )
        def sc_gather(table_hbm, idx_hbm, out_hbm, idx_v, rows_v, sem):
            wid = lax.axis_index("s") * _NUM_CORES + lax.axis_index("c")
            base = wid * _BPW
            pltpu.sync_copy(idx_hbm.at[pl.ds(base, _BPW)], idx_v)
            pltpu.async_copy(table_hbm.at[idx_v], rows_v, sem).wait()
            pltpu.sync_copy(rows_v, out_hbm.at[pl.ds(base, _BPW)])

        _sc_gather_fn = sc_gather
    return _sc_gather_fn


def _proj_body(emb_ref, w_ref, b_ref, out_hbm, x_ref, obufs, tbuf, *sems):
    i = pl.program_id(0)
    tail_sem = sems[_NBUF]

    @pl.when(i == 0)
    def _():
        emb = emb_ref[...]
        norm = jnp.sqrt(jnp.sum(emb * emb, axis=1, keepdims=True))
        scale = jnp.minimum(1.0, _MAX_NORM / jnp.maximum(norm, 1e-7))
        x_ref[...] = emb * scale

    result = lax.dot_general(
        x_ref[...], w_ref[...],
        (((1,), (1,)), ((), ())),
        preferred_element_type=jnp.float32,
    ) + b_ref[...]

    slot = lax.rem(i, _NBUF)

    @pl.when(i < _GRID - 1)
    def _():
        for s in range(_NBUF):
            @pl.when(slot == s)
            def _(s=s):
                @pl.when(i >= _NBUF)
                def _():
                    pltpu.make_async_copy(
                        obufs.at[s], out_hbm.at[:, pl.ds(0, _V_BLK)], sems[s]
                    ).wait()
                obufs[s, ...] = result
                for r in range(_NCHUNK):
                    pltpu.make_async_copy(
                        obufs.at[s, pl.ds(r * _ROWS, _ROWS)],
                        out_hbm.at[pl.ds(r * _ROWS, _ROWS),
                                   pl.ds(i * _V_BLK, _V_BLK)],
                        sems[s],
                    ).start(priority=(r + s) % _NTHREAD)

    @pl.when(i == _GRID - 1)
    def _():
        tbuf[...] = result[:, :_TAIL]
        pltpu.make_async_copy(
            tbuf, out_hbm.at[:, pl.ds((_GRID - 1) * _V_BLK, _TAIL)], tail_sem
        ).start()
        for s in range(_NBUF):
            pltpu.make_async_copy(
                obufs.at[s], out_hbm.at[:, pl.ds(0, _V_BLK)], sems[s]
            ).wait()
        pltpu.make_async_copy(
            tbuf, out_hbm.at[:, pl.ds((_GRID - 1) * _V_BLK, _TAIL)], tail_sem
        ).wait()


def _projection(emb, W, b2):
    return pl.pallas_call(
        _proj_body,
        grid=(_GRID,),
        in_specs=[
            pl.BlockSpec((_BATCH, _DIM), lambda i: (0, 0)),
            pl.BlockSpec((_V_BLK, _DIM), lambda i: (i, 0)),
            pl.BlockSpec((1, _V_BLK), lambda i: (0, i)),
        ],
        out_specs=pl.BlockSpec(memory_space=pl.ANY),
        out_shape=jax.ShapeDtypeStruct((_BATCH, _VOCAB), jnp.float32),
        scratch_shapes=[
            pltpu.VMEM((_BATCH, _DIM), jnp.float32),
            pltpu.VMEM((_NBUF, _BATCH, _V_BLK), jnp.float32),
            pltpu.VMEM((_BATCH, _TAIL), jnp.float32),
        ] + [pltpu.SemaphoreType.DMA] * (_NBUF + 1),
    )(emb, W, b2)


def kernel(inputs_, table, W, b):
    emb = _get_sc_gather()(table, inputs_)
    return _projection(emb, W, b.reshape(1, _VOCAB))


# speedup vs baseline: 1.9041x; 1.9041x over previous
import functools

import jax
import jax.numpy as jnp
from jax import lax
from jax.experimental import pallas as pl
from jax.experimental.pallas import tpu as pltpu
from jax.experimental.pallas import tpu_sc as plsc

_VOCAB = 100000
_DIM = 64
_BATCH = 1024
_MAX_NORM = 1.0

_NUM_CORES = 2
_NUM_SUBCORES = 16
_NW = _NUM_CORES * _NUM_SUBCORES
_BPW = _BATCH // _NW

_V_BLK = 2000
_GRID = _VOCAB // _V_BLK

_sc_gather_fn = None


def _get_sc_gather():
    global _sc_gather_fn
    if _sc_gather_fn is None:
        mesh = plsc.VectorSubcoreMesh(core_axis_name="c", subcore_axis_name="s")

        @functools.partial(
            pl.kernel,
            mesh=mesh,
            compiler_params=pltpu.CompilerParams(use_tc_tiling_on_sc=False),
            out_type=jax.ShapeDtypeStruct((_BATCH, _DIM), jnp.float32),
            scratch_types=[
                pltpu.VMEM((_BPW,), jnp.int32),
                pltpu.VMEM((_BPW, _DIM), jnp.float32),
                pltpu.SemaphoreType.DMA,
            ],
        )
        def sc_gather(table_hbm, idx_hbm, out_hbm, idx_v, rows_v, sem):
            wid = lax.axis_index("s") * _NUM_CORES + lax.axis_index("c")
            base = wid * _BPW
            pltpu.sync_copy(idx_hbm.at[pl.ds(base, _BPW)], idx_v)
            pltpu.async_copy(table_hbm.at[idx_v], rows_v, sem).wait()
            pltpu.sync_copy(rows_v, out_hbm.at[pl.ds(base, _BPW)])

        _sc_gather_fn = sc_gather
    return _sc_gather_fn


def _proj_body(emb_ref, w_ref, b_ref, out_ref, x_ref):
    @pl.when(pl.program_id(0) == 0)
    def _():
        emb = emb_ref[...]
        norm = jnp.sqrt(jnp.sum(emb * emb, axis=1, keepdims=True))
        scale = jnp.minimum(1.0, _MAX_NORM / jnp.maximum(norm, 1e-7))
        x_ref[...] = emb * scale

    out_ref[...] = lax.dot_general(
        w_ref[...], x_ref[...],
        (((1,), (1,)), ((), ())),
        preferred_element_type=jnp.float32,
    ) + b_ref[...]


def _projection_t(emb, W, b_col):
    return pl.pallas_call(
        _proj_body,
        grid=(_GRID,),
        in_specs=[
            pl.BlockSpec((_BATCH, _DIM), lambda i: (0, 0)),
            pl.BlockSpec((_V_BLK, _DIM), lambda i: (i, 0)),
            pl.BlockSpec((_V_BLK, 1), lambda i: (i, 0)),
        ],
        out_specs=pl.BlockSpec((_V_BLK, _BATCH), lambda i: (i, 0)),
        out_shape=jax.ShapeDtypeStruct((_VOCAB, _BATCH), jnp.float32),
        scratch_shapes=[pltpu.VMEM((_BATCH, _DIM), jnp.float32)],
    )(emb, W, b_col)


def kernel(inputs_, table, W, b):
    emb = _get_sc_gather()(table, inputs_)
    out_t = _projection_t(emb, W, b.reshape(_VOCAB, 1))
    return out_t.T


# speedup vs baseline: 2.2161x; 1.1638x over previous
import functools

import jax
import jax.numpy as jnp
from jax import lax
from jax.experimental import pallas as pl
from jax.experimental.pallas import tpu as pltpu
from jax.experimental.pallas import tpu_sc as plsc

_VOCAB = 100000
_DIM = 64
_BATCH = 1024
_MAX_NORM = 1.0

_NUM_CORES = 2
_NUM_SUBCORES = 16
_NW = _NUM_CORES * _NUM_SUBCORES
_BPW = _BATCH // _NW

_V_BLK = 2048
_GRID = (_VOCAB + _V_BLK - 1) // _V_BLK

_sc_gather_fn = None


def _get_sc_gather():
    global _sc_gather_fn
    if _sc_gather_fn is None:
        mesh = plsc.VectorSubcoreMesh(core_axis_name="c", subcore_axis_name="s")

        @functools.partial(
            pl.kernel,
            mesh=mesh,
            compiler_params=pltpu.CompilerParams(use_tc_tiling_on_sc=False),
            out_type=jax.ShapeDtypeStruct((_BATCH, _DIM), jnp.float32),
            scratch_types=[
                pltpu.VMEM((_BPW,), jnp.int32),
                pltpu.VMEM((_BPW, _DIM), jnp.float32),
                pltpu.SemaphoreType.DMA,
            ],
        )
        def sc_gather(table_hbm, idx_hbm, out_hbm, idx_v, rows_v, sem):
            wid = lax.axis_index("s") * _NUM_CORES + lax.axis_index("c")
            base = wid * _BPW
            pltpu.sync_copy(idx_hbm.at[pl.ds(base, _BPW)], idx_v)
            pltpu.async_copy(table_hbm.at[idx_v], rows_v, sem).wait()
            pltpu.sync_copy(rows_v, out_hbm.at[pl.ds(base, _BPW)])

        _sc_gather_fn = sc_gather
    return _sc_gather_fn


def _proj_body(emb_ref, w_ref, b_ref, out_ref, x_ref):
    @pl.when(pl.program_id(0) == 0)
    def _():
        emb = emb_ref[...]
        norm = jnp.sqrt(jnp.sum(emb * emb, axis=1, keepdims=True))
        scale = jnp.minimum(1.0, _MAX_NORM / jnp.maximum(norm, 1e-7))
        x_ref[...] = emb * scale

    out_ref[...] = lax.dot_general(
        w_ref[...], x_ref[...],
        (((0,), (1,)), ((), ())),
        preferred_element_type=jnp.float32,
    ) + b_ref[...]


def _projection_t(emb, W_t, b_col):
    return pl.pallas_call(
        _proj_body,
        grid=(_GRID,),
        in_specs=[
            pl.BlockSpec((_BATCH, _DIM), lambda i: (0, 0)),
            pl.BlockSpec((_DIM, _V_BLK), lambda i: (0, i)),
            pl.BlockSpec((_V_BLK, 1), lambda i: (i, 0)),
        ],
        out_specs=pl.BlockSpec((_V_BLK, _BATCH), lambda i: (i, 0)),
        out_shape=jax.ShapeDtypeStruct((_VOCAB, _BATCH), jnp.float32),
        scratch_shapes=[pltpu.VMEM((_BATCH, _DIM), jnp.float32)],
    )(emb, W_t, b_col)


def kernel(inputs_, table, W, b):
    emb = _get_sc_gather()(table, inputs_)
    out_t = _projection_t(emb, W.T, b.reshape(_VOCAB, 1))
    return out_t.T


# speedup vs baseline: 2.2858x; 1.0315x over previous
import functools

import jax
import jax.numpy as jnp
from jax import lax
from jax.experimental import pallas as pl
from jax.experimental.pallas import tpu as pltpu
from jax.experimental.pallas import tpu_sc as plsc

_VOCAB = 100000
_DIM = 64
_BATCH = 1024
_MAX_NORM = 1.0

_NUM_CORES = 2
_NUM_SUBCORES = 16
_NW = _NUM_CORES * _NUM_SUBCORES
_BPW = _BATCH // _NW

_V_BLK = 4096
_GRID = (_VOCAB + _V_BLK - 1) // _V_BLK

_sc_gather_fn = None


def _get_sc_gather():
    global _sc_gather_fn
    if _sc_gather_fn is None:
        mesh = plsc.VectorSubcoreMesh(core_axis_name="c", subcore_axis_name="s")

        @functools.partial(
            pl.kernel,
            mesh=mesh,
            compiler_params=pltpu.CompilerParams(use_tc_tiling_on_sc=False),
            out_type=jax.ShapeDtypeStruct((_BATCH, _DIM), jnp.float32),
            scratch_types=[
                pltpu.VMEM((_BPW,), jnp.int32),
                pltpu.VMEM((_BPW, _DIM), jnp.float32),
                pltpu.SemaphoreType.DMA,
            ],
        )
        def sc_gather(table_hbm, idx_hbm, out_hbm, idx_v, rows_v, sem):
            wid = lax.axis_index("s") * _NUM_CORES + lax.axis_index("c")
            base = wid * _BPW
            pltpu.sync_copy(idx_hbm.at[pl.ds(base, _BPW)], idx_v)
            pltpu.async_copy(table_hbm.at[idx_v], rows_v, sem).wait()
            pltpu.sync_copy(rows_v, out_hbm.at[pl.ds(base, _BPW)])

        _sc_gather_fn = sc_gather
    return _sc_gather_fn


def _proj_body(emb_ref, w_ref, b_ref, out_ref, x_ref):
    @pl.when(pl.program_id(0) == 0)
    def _():
        emb = emb_ref[...]
        norm = jnp.sqrt(jnp.sum(emb * emb, axis=1, keepdims=True))
        scale = jnp.minimum(1.0, _MAX_NORM / jnp.maximum(norm, 1e-7))
        x_ref[...] = emb * scale

    out_ref[...] = lax.dot_general(
        w_ref[...], x_ref[...],
        (((0,), (1,)), ((), ())),
        preferred_element_type=jnp.float32,
    ) + b_ref[...]


def _projection_t(emb, W_t, b_col):
    return pl.pallas_call(
        _proj_body,
        grid=(_GRID,),
        in_specs=[
            pl.BlockSpec((_BATCH, _DIM), lambda i: (0, 0)),
            pl.BlockSpec((_DIM, _V_BLK), lambda i: (0, i)),
            pl.BlockSpec((_V_BLK, 1), lambda i: (i, 0)),
        ],
        out_specs=pl.BlockSpec((_V_BLK, _BATCH), lambda i: (i, 0)),
        out_shape=jax.ShapeDtypeStruct((_VOCAB, _BATCH), jnp.float32),
        scratch_shapes=[pltpu.VMEM((_BATCH, _DIM), jnp.float32)],
    )(emb, W_t, b_col)


def kernel(inputs_, table, W, b):
    emb = _get_sc_gather()(table, inputs_)
    out_t = _projection_t(emb, W.T, b.reshape(_VOCAB, 1))
    return out_t.T
